# baseline (speedup 1.0000x reference)
import jax
import jax.numpy as jnp
from jax.experimental import pallas as pl

_BLOCK = 128


def _body(x_ref, w_ref, x0_ref, x1_ref, out_ref):
    x = x_ref[...]
    logits = jnp.dot(x, w_ref[...], preferred_element_type=jnp.float32)
    gates = jax.nn.softmax(logits, axis=-1)
    g0 = gates[:, 0:1]
    g1 = gates[:, 1:2]
    top0 = g0 >= g1
    s0 = jnp.where(top0, g0, 0.0)
    s1 = jnp.where(top0, 0.0, g1)
    x0 = x * s0
    x1 = x * s1
    x0_ref[...] = x0
    x1_ref[...] = x1
    out_ref[...] = x0 + x1


def kernel(x, W_gate):
    n, d = x.shape
    row_spec = pl.BlockSpec((_BLOCK, d), lambda i: (i, 0))
    w_spec = pl.BlockSpec(W_gate.shape, lambda i: (0, 0))
    out_sds = jax.ShapeDtypeStruct((n, d), x.dtype)
    x0, x1, out = pl.pallas_call(
        _body,
        grid=(n // _BLOCK,),
        in_specs=[row_spec, w_spec],
        out_specs=[row_spec, row_spec, row_spec],
        out_shape=(out_sds, out_sds, out_sds),
    )(x, W_gate)
    return (x0, x1, out)


# speedup vs baseline: 1.0493x; 1.0493x over previous
import jax
import jax.numpy as jnp
from jax.experimental import pallas as pl
from jax.experimental.pallas import tpu as pltpu

_BLOCK = 256


def _body(x_ref, w_ref, x0_ref, x1_ref, out_ref):
    x = x_ref[...]
    logits = jnp.dot(x, w_ref[...], preferred_element_type=jnp.float32)
    gates = jax.nn.softmax(logits, axis=-1)
    g0 = gates[:, 0:1]
    g1 = gates[:, 1:2]
    top0 = g0 >= g1
    s0 = jnp.where(top0, g0, 0.0)
    s1 = jnp.where(top0, 0.0, g1)
    x0 = x * s0
    x1 = x * s1
    x0_ref[...] = x0
    x1_ref[...] = x1
    out_ref[...] = x0 + x1


def kernel(x, W_gate):
    n, d = x.shape
    row_spec = pl.BlockSpec((_BLOCK, d), lambda i: (i, 0))
    w_spec = pl.BlockSpec(W_gate.shape, lambda i: (0, 0))
    out_sds = jax.ShapeDtypeStruct((n, d), x.dtype)
    x0, x1, out = pl.pallas_call(
        _body,
        grid=(n // _BLOCK,),
        in_specs=[row_spec, w_spec],
        out_specs=[row_spec, row_spec, row_spec],
        out_shape=(out_sds, out_sds, out_sds),
        compiler_params=pltpu.CompilerParams(
            dimension_semantics=("arbitrary",),
            vmem_limit_bytes=120 * 1024 * 1024,
        ),
    )(x, W_gate)
    return (x0, x1, out)


# speedup vs baseline: 1.0497x; 1.0003x over previous
import jax
import jax.numpy as jnp
from jax.experimental import pallas as pl

_BLOCK = 256


def _body(x_ref, w_ref, x0_ref, x1_ref, out_ref):
    x = x_ref[...]
    logits = jnp.dot(x, w_ref[...], preferred_element_type=jnp.float32)
    gates = jax.nn.softmax(logits, axis=-1)
    g0 = gates[:, 0:1]
    g1 = gates[:, 1:2]
    top0 = g0 >= g1
    s0 = jnp.where(top0, g0, 0.0)
    s1 = jnp.where(top0, 0.0, g1)
    x0 = x * s0
    x1 = x * s1
    x0_ref[...] = x0
    x1_ref[...] = x1
    out_ref[...] = x0 + x1


def kernel(x, W_gate):
    n, d = x.shape
    row_spec = pl.BlockSpec((_BLOCK, d), lambda i: (i, 0))
    w_spec = pl.BlockSpec(W_gate.shape, lambda i: (0, 0))
    out_sds = jax.ShapeDtypeStruct((n, d), x.dtype)
    x0, x1, out = pl.pallas_call(
        _body,
        grid=(n // _BLOCK,),
        in_specs=[row_spec, w_spec],
        out_specs=[row_spec, row_spec, row_spec],
        out_shape=(out_sds, out_sds, out_sds),
    )(x, W_gate)
    return (x0, x1, out)
